# Initial kernel scaffold; baseline (speedup 1.0000x reference)
import jax
import jax.numpy as jnp
from jax.experimental import pallas as pl


def kernel(x, y, table0, table1, table2, table3, table4, table5, table6, table7, table8, W, b):
    raise NotImplementedError("write your pallas kernel here")


# speedup vs baseline: 2.1768x; 2.1768x over previous
import functools

import jax
import jax.numpy as jnp
from jax import lax
from jax.experimental import pallas as pl
from jax.experimental.pallas import tpu as pltpu
from jax.experimental.pallas import tpu_sc as plsc

B, T, C = 1024, 200, 7
N_EMBD = 32
LAG = 60
SIZES = (13, 32, 24, 7, 200, 2, 61, 2, 3)
OFFS = (0, 13, 45, 69, 76, 276, 278, 339, 341)
TOTAL_ROWS = 344

NC, NS, L = 2, 16, 16
NW = NC * NS
ROWS_PER_W = B // NW
TPAD = 208
NG = TPAD // L


def _splat(v):
    return jnp.full((L,), v, jnp.int32)


def _sc_body(x_hbm, y_hbm, tab_hbm, w_hbm, bias_hbm, out_hbm,
             tab_v, x_v, y_v, w_v, bias_v, out_v):
    wid = lax.axis_index("s") * NC + lax.axis_index("c")
    base = wid * ROWS_PER_W

    pltpu.sync_copy(tab_hbm, tab_v)
    pltpu.sync_copy(w_hbm, w_v)
    pltpu.sync_copy(bias_hbm, bias_v)

    @pl.loop(0, ROWS_PER_W)
    def _row(j):
        bi = base + j
        pltpu.sync_copy(x_hbm.at[bi], x_v.at[pl.ds(0, T)])
        pltpu.sync_copy(y_hbm.at[bi], y_v.at[pl.ds(0, T)])

        @pl.loop(0, NG)
        def _grp(g):
            t0 = g * L
            tvec = t0 + lax.iota(jnp.int32, L)

            for p in range(6):
                raw = plsc.load_gather(x_v, [tvec, _splat(p + 1)])
                row = jnp.clip(raw, 0, SIZES[p] - 1) + OFFS[p]
                for col in range(N_EMBD):
                    v = plsc.load_gather(tab_v, [row, _splat(col)])
                    plsc.store_scatter(
                        out_v, [tvec, _splat(p * N_EMBD + col)], v)

            r6 = jnp.minimum(tvec, SIZES[6] - 1) + OFFS[6]
            isfut = (tvec >= (T - LAG)).astype(jnp.int32)
            r7 = isfut + OFFS[7]
            r8 = isfut + OFFS[8]
            for p, row in ((6, r6), (7, r7), (8, r8)):
                for col in range(N_EMBD):
                    v = plsc.load_gather(tab_v, [row, _splat(col)])
                    plsc.store_scatter(
                        out_v, [tvec, _splat(p * N_EMBD + col)], v)

            yvec = y_v[pl.ds(t0, L)]
            for col in range(N_EMBD):
                wv = plsc.load_gather(w_v, [_splat(col)])
                bv = plsc.load_gather(bias_v, [_splat(col)])
                plsc.store_scatter(
                    out_v, [tvec, _splat(9 * N_EMBD + col)], yvec * wv + bv)

        pltpu.sync_copy(out_v.at[pl.ds(0, T)], out_hbm.at[bi])


@jax.jit
def _run(x, y2, big_table, w_row, bias):
    mesh = plsc.VectorSubcoreMesh(
        core_axis_name="c", subcore_axis_name="s",
        num_cores=NC, num_subcores=NS)
    f = pl.kernel(
        _sc_body,
        out_type=jax.ShapeDtypeStruct((B, T, 10 * N_EMBD), jnp.float32),
        mesh=mesh,
        compiler_params=pltpu.CompilerParams(
            needs_layout_passes=False, use_tc_tiling_on_sc=False),
        scratch_types=[
            pltpu.VMEM((TOTAL_ROWS, N_EMBD), jnp.float32),
            pltpu.VMEM((TPAD, C), jnp.int32),
            pltpu.VMEM((TPAD,), jnp.float32),
            pltpu.VMEM((N_EMBD,), jnp.float32),
            pltpu.VMEM((N_EMBD,), jnp.float32),
            pltpu.VMEM((TPAD, 10 * N_EMBD), jnp.float32),
        ],
    )
    return f(x, y2, big_table, w_row, bias)


def kernel(x, y, table0, table1, table2, table3, table4, table5, table6,
           table7, table8, W, b):
    big_table = jnp.concatenate(
        [table0, table1, table2, table3, table4, table5, table6, table7,
         table8], axis=0)
    return _run(x, y[:, :, 0], big_table, W[0], b)


# speedup vs baseline: 2.9613x; 1.3604x over previous
import jax
import jax.numpy as jnp
from jax import lax
from jax.experimental import pallas as pl
from jax.experimental.pallas import tpu as pltpu
from jax.experimental.pallas import tpu_sc as plsc

B, T, C = 1024, 200, 7
N_EMBD = 32
LAG = 60
SIZES = (13, 32, 24, 7, 200, 2, 61, 2, 3)
OFFS = (0, 13, 45, 69, 76, 276, 278, 339, 341)
TOTAL_ROWS = 344
D = 10 * N_EMBD

NC, NS, L = 2, 16, 16
NW = NC * NS
ROWS_PER_W = B // NW
TPAD = 208
NG = TPAD // L
NG_LO = 7
T_LO = NG_LO * L


def _splat(v):
    return jnp.full((L,), v, jnp.int32)


def _sc_body(x_hbm, y_hbm, tab_hbm, w_hbm, bias_hbm, out_hbm,
             tab_v, x_v, y_v, w_v, bias_v, out_v, sem_lo, sem_hi):
    wid = lax.axis_index("s") * NC + lax.axis_index("c")
    base_row = wid * ROWS_PER_W

    pltpu.sync_copy(tab_hbm, tab_v)
    pltpu.sync_copy(w_hbm, w_v)
    pltpu.sync_copy(bias_hbm, bias_v)

    @pl.loop(0, NG)
    def _const(g):
        tvec = g * L + lax.iota(jnp.int32, L)
        obase = tvec * D
        r6 = (jnp.minimum(tvec, SIZES[6] - 1) + OFFS[6]) * N_EMBD
        isfut = (tvec >= (T - LAG)).astype(jnp.int32)
        r7 = (isfut + OFFS[7]) * N_EMBD
        r8 = (isfut + OFFS[8]) * N_EMBD
        for p, row in ((6, r6), (7, r7), (8, r8)):
            for col in range(N_EMBD):
                v = plsc.load_gather(tab_v, [row + _splat(col)])
                plsc.store_scatter(
                    out_v, [obase + _splat(p * N_EMBD + col)], v)

    def _compute_groups(bi, g_lo, g_hi):
        @pl.loop(g_lo, g_hi)
        def _grp(g):
            t0 = g * L
            tvec = t0 + lax.iota(jnp.int32, L)
            obase = tvec * D
            xbase = tvec * C

            for p in range(6):
                raw = plsc.load_gather(x_v, [xbase + _splat(p + 1)])
                row = (jnp.clip(raw, 0, SIZES[p] - 1) + OFFS[p]) * N_EMBD
                for col in range(N_EMBD):
                    v = plsc.load_gather(tab_v, [row + _splat(col)])
                    plsc.store_scatter(
                        out_v, [obase + _splat(p * N_EMBD + col)], v)

            yvec = y_v[pl.ds(t0, L)]
            for col in range(N_EMBD):
                wv = plsc.load_gather(w_v, [_splat(col)])
                bv = plsc.load_gather(bias_v, [_splat(col)])
                plsc.store_scatter(
                    out_v, [obase + _splat(9 * N_EMBD + col)],
                    yvec * wv + bv)

    n_lo = T_LO * D
    n_hi = (T - T_LO) * D

    def _dma_lo(bi):
        return pltpu.make_async_copy(
            out_v.at[pl.ds(0, n_lo)],
            out_hbm.at[bi, pl.ds(0, n_lo)], sem_lo)

    def _dma_hi(bi):
        return pltpu.make_async_copy(
            out_v.at[pl.ds(n_lo, n_hi)],
            out_hbm.at[bi, pl.ds(n_lo, n_hi)], sem_hi)

    @pl.loop(0, ROWS_PER_W)
    def _row(j):
        bi = base_row + j
        pltpu.sync_copy(x_hbm.at[bi], x_v.at[pl.ds(0, T * C)])
        pltpu.sync_copy(y_hbm.at[bi], y_v.at[pl.ds(0, T)])

        @pl.when(j > 0)
        def _():
            _dma_lo(bi).wait()

        _compute_groups(bi, 0, NG_LO)
        _dma_lo(bi).start()

        @pl.when(j > 0)
        def _():
            _dma_hi(bi).wait()

        _compute_groups(bi, NG_LO, NG)
        _dma_hi(bi).start()

    _dma_lo(base_row + ROWS_PER_W - 1).wait()
    _dma_hi(base_row + ROWS_PER_W - 1).wait()


@jax.jit
def _run(x2, y2, tab_flat, w_row, bias):
    mesh = plsc.VectorSubcoreMesh(
        core_axis_name="c", subcore_axis_name="s",
        num_cores=NC, num_subcores=NS)
    f = pl.kernel(
        _sc_body,
        out_type=jax.ShapeDtypeStruct((B, T * D), jnp.float32),
        mesh=mesh,
        compiler_params=pltpu.CompilerParams(
            needs_layout_passes=False, use_tc_tiling_on_sc=False),
        scratch_types=[
            pltpu.VMEM((TOTAL_ROWS * N_EMBD,), jnp.float32),
            pltpu.VMEM((TPAD * C,), jnp.int32),
            pltpu.VMEM((TPAD,), jnp.float32),
            pltpu.VMEM((N_EMBD,), jnp.float32),
            pltpu.VMEM((N_EMBD,), jnp.float32),
            pltpu.VMEM((TPAD * D,), jnp.float32),
            pltpu.SemaphoreType.DMA,
            pltpu.SemaphoreType.DMA,
        ],
    )
    return f(x2, y2, tab_flat, w_row, bias)


def kernel(x, y, table0, table1, table2, table3, table4, table5, table6,
           table7, table8, W, b):
    tab_flat = jnp.concatenate(
        [table0, table1, table2, table3, table4, table5, table6, table7,
         table8], axis=0).reshape(-1)
    out = _run(x.reshape(B, T * C), y[:, :, 0], tab_flat, W[0], b)
    return out.reshape(B, T, D)


# speedup vs baseline: 6.0182x; 2.0323x over previous
import jax
import jax.numpy as jnp
from jax import lax
from jax.experimental import pallas as pl
from jax.experimental.pallas import tpu as pltpu
from jax.experimental.pallas import tpu_sc as plsc

B, T, C = 1024, 200, 7
N_EMBD = 32
LAG = 60
SIZES = (13, 32, 24, 7, 200, 2, 61, 2, 3)
OFFS = (0, 13, 45, 69, 76, 276, 278, 339, 341)
TOTAL_ROWS = 344
D = 10 * N_EMBD
TABW = N_EMBD + 1
DPAD = D + 9

NC, NS, L = 2, 16, 16
NW = NC * NS
ROWS_PER_W = B // NW
TPAD = 208
NG = TPAD // L
NG_LO = 7
T_LO = NG_LO * L


def _splat(v):
    return jnp.full((L,), v, jnp.int32)


def _sc_body(x_hbm, y_hbm, tab_hbm, wb_hbm, out_hbm,
             tab_v, x_v, y_v, wb_v, out_v, sem_lo, sem_hi):
    wid = lax.axis_index("s") * NC + lax.axis_index("c")
    base_row = wid * ROWS_PER_W

    pltpu.sync_copy(tab_hbm, tab_v)
    pltpu.sync_copy(wb_hbm, wb_v)

    @pl.loop(0, NG)
    def _const(g):
        tvec = g * L + lax.iota(jnp.int32, L)
        r6 = (jnp.minimum(tvec, SIZES[6] - 1) + OFFS[6]) * TABW
        isfut = (tvec >= (T - LAG)).astype(jnp.int32)
        r7 = (isfut + OFFS[7]) * TABW
        r8 = (isfut + OFFS[8]) * TABW
        for p, row in ((6, r6), (7, r7), (8, r8)):
            for col in range(N_EMBD):
                v = plsc.load_gather(tab_v, [row + _splat(col)])
                plsc.store_scatter(
                    out_v, [tvec, _splat(p * N_EMBD + col)], v)

    def _compute_groups(g_lo, g_hi):
        @pl.loop(g_lo, g_hi)
        def _grp(g):
            t0 = g * L
            tvec = t0 + lax.iota(jnp.int32, L)
            xbase = tvec * C

            for p in range(6):
                raw = plsc.load_gather(x_v, [xbase + _splat(p + 1)])
                row = (jnp.clip(raw, 0, SIZES[p] - 1) + OFFS[p]) * TABW
                for col in range(N_EMBD):
                    v = plsc.load_gather(tab_v, [row + _splat(col)])
                    plsc.store_scatter(
                        out_v, [tvec, _splat(p * N_EMBD + col)], v)

            yvec = y_v[pl.ds(t0, L)]
            for col in range(N_EMBD):
                wv = wb_v[pl.ds(col * L, L)]
                bv = wb_v[pl.ds((N_EMBD + col) * L, L)]
                plsc.store_scatter(
                    out_v, [tvec, _splat(9 * N_EMBD + col)],
                    yvec * wv + bv)

    def _dma_lo(bi):
        return pltpu.make_async_copy(
            out_v.at[pl.ds(0, T_LO), pl.ds(0, D)],
            out_hbm.at[bi, pl.ds(0, T_LO)], sem_lo)

    def _dma_hi(bi):
        return pltpu.make_async_copy(
            out_v.at[pl.ds(T_LO, T - T_LO), pl.ds(0, D)],
            out_hbm.at[bi, pl.ds(T_LO, T - T_LO)], sem_hi)

    @pl.loop(0, ROWS_PER_W)
    def _row(j):
        bi = base_row + j
        pltpu.sync_copy(x_hbm.at[bi], x_v.at[pl.ds(0, T * C)])
        pltpu.sync_copy(y_hbm.at[bi], y_v.at[pl.ds(0, T)])

        @pl.when(j > 0)
        def _():
            _dma_lo(bi).wait()

        _compute_groups(0, NG_LO)
        _dma_lo(bi).start()

        @pl.when(j > 0)
        def _():
            _dma_hi(bi).wait()

        _compute_groups(NG_LO, NG)
        _dma_hi(bi).start()

    _dma_lo(base_row + ROWS_PER_W - 1).wait()
    _dma_hi(base_row + ROWS_PER_W - 1).wait()


@jax.jit
def _run(x2, y2, tab_pad, wb_rep):
    mesh = plsc.VectorSubcoreMesh(
        core_axis_name="c", subcore_axis_name="s",
        num_cores=NC, num_subcores=NS)
    f = pl.kernel(
        _sc_body,
        out_type=jax.ShapeDtypeStruct((B, T, D), jnp.float32),
        mesh=mesh,
        compiler_params=pltpu.CompilerParams(
            needs_layout_passes=False, use_tc_tiling_on_sc=False),
        scratch_types=[
            pltpu.VMEM((TOTAL_ROWS * TABW,), jnp.float32),
            pltpu.VMEM((TPAD * C,), jnp.int32),
            pltpu.VMEM((TPAD,), jnp.float32),
            pltpu.VMEM((2 * N_EMBD * L,), jnp.float32),
            pltpu.VMEM((TPAD, DPAD), jnp.float32),
            pltpu.SemaphoreType.DMA,
            pltpu.SemaphoreType.DMA,
        ],
    )
    return f(x2, y2, tab_pad, wb_rep)


def kernel(x, y, table0, table1, table2, table3, table4, table5, table6,
           table7, table8, W, b):
    tab = jnp.concatenate(
        [table0, table1, table2, table3, table4, table5, table6, table7,
         table8], axis=0)
    tab_pad = jnp.pad(tab, ((0, 0), (0, TABW - N_EMBD))).reshape(-1)
    wb_rep = jnp.concatenate([
        jnp.repeat(W[0], L), jnp.repeat(b, L)])
    return _run(x.reshape(B, T * C), y[:, :, 0], tab_pad, wb_rep)


# speedup vs baseline: 8.0404x; 1.3360x over previous
import jax
import jax.numpy as jnp
from jax import lax
from jax.experimental import pallas as pl
from jax.experimental.pallas import tpu as pltpu
from jax.experimental.pallas import tpu_sc as plsc

B, T, C = 1024, 200, 7
N_EMBD = 32
LAG = 60
SIZES = (13, 32, 24, 7, 200, 2, 61, 2, 3)
OFFS = (0, 13, 45, 69, 76, 276, 278, 339, 341)
TOTAL_ROWS = 344
D = 10 * N_EMBD
TABW = N_EMBD + 1
DPAD = D + 9

NC, NS, L = 2, 16, 16
NW = NC * NS
ROWS_PER_W = B // NW
TPAD = 208
NG = TPAD // L
NG_LO = 7
T_LO = NG_LO * L


def _splat(v):
    return jnp.full((L,), v, jnp.int32)


def _sc_body(x_hbm, y_hbm, tab_hbm, wb_hbm, out_hbm,
             tab_v, x_v, y_v, wb_v, out_v, sem_lo, sem_hi):
    wid = lax.axis_index("s") * NC + lax.axis_index("c")
    base_row = wid * ROWS_PER_W

    pltpu.sync_copy(tab_hbm, tab_v)
    pltpu.sync_copy(wb_hbm, wb_v)

    @pl.loop(0, NG)
    def _const(g):
        tvec = g * L + lax.iota(jnp.int32, L)
        r6 = (jnp.minimum(tvec, SIZES[6] - 1) + OFFS[6]) * TABW
        isfut = (tvec >= (T - LAG)).astype(jnp.int32)
        r7 = (isfut + OFFS[7]) * TABW
        r8 = (isfut + OFFS[8]) * TABW
        for p, row in ((6, r6), (7, r7), (8, r8)):
            vals = [plsc.load_gather(tab_v, [row + _splat(col)])
                    for col in range(N_EMBD)]
            for col in range(N_EMBD):
                plsc.store_scatter(
                    out_v, [tvec, _splat(p * N_EMBD + col)], vals[col])

    def _compute_groups(g_lo, g_hi):
        @pl.loop(g_lo, g_hi)
        def _grp(g):
            t0 = g * L
            tvec = t0 + lax.iota(jnp.int32, L)
            xbase = tvec * C

            raws = [plsc.load_gather(x_v, [xbase + _splat(p + 1)])
                    for p in range(6)]
            rows = [(jnp.clip(raws[p], 0, SIZES[p] - 1) + OFFS[p]) * TABW
                    for p in range(6)]
            for p in range(6):
                vals = [plsc.load_gather(tab_v, [rows[p] + _splat(col)])
                        for col in range(N_EMBD)]
                for col in range(N_EMBD):
                    plsc.store_scatter(
                        out_v, [tvec, _splat(p * N_EMBD + col)],
                        vals[col])

            yvec = y_v[pl.ds(t0, L)]
            lins = [yvec * wb_v[pl.ds(col * L, L)]
                    + wb_v[pl.ds((N_EMBD + col) * L, L)]
                    for col in range(N_EMBD)]
            for col in range(N_EMBD):
                plsc.store_scatter(
                    out_v, [tvec, _splat(9 * N_EMBD + col)], lins[col])

    def _dma_lo(bi):
        return pltpu.make_async_copy(
            out_v.at[pl.ds(0, T_LO), pl.ds(0, D)],
            out_hbm.at[bi, pl.ds(0, T_LO)], sem_lo)

    def _dma_hi(bi):
        return pltpu.make_async_copy(
            out_v.at[pl.ds(T_LO, T - T_LO), pl.ds(0, D)],
            out_hbm.at[bi, pl.ds(T_LO, T - T_LO)], sem_hi)

    @pl.loop(0, ROWS_PER_W)
    def _row(j):
        bi = base_row + j
        pltpu.sync_copy(x_hbm.at[bi], x_v.at[pl.ds(0, T * C)])
        pltpu.sync_copy(y_hbm.at[bi], y_v.at[pl.ds(0, T)])

        @pl.when(j > 0)
        def _():
            _dma_lo(bi).wait()

        _compute_groups(0, NG_LO)
        _dma_lo(bi).start()

        @pl.when(j > 0)
        def _():
            _dma_hi(bi).wait()

        _compute_groups(NG_LO, NG)
        _dma_hi(bi).start()

    _dma_lo(base_row + ROWS_PER_W - 1).wait()
    _dma_hi(base_row + ROWS_PER_W - 1).wait()


@jax.jit
def _run(x2, y2, tab_pad, wb_rep):
    mesh = plsc.VectorSubcoreMesh(
        core_axis_name="c", subcore_axis_name="s",
        num_cores=NC, num_subcores=NS)
    f = pl.kernel(
        _sc_body,
        out_type=jax.ShapeDtypeStruct((B, T, D), jnp.float32),
        mesh=mesh,
        compiler_params=pltpu.CompilerParams(
            needs_layout_passes=False, use_tc_tiling_on_sc=False),
        scratch_types=[
            pltpu.VMEM((TOTAL_ROWS * TABW,), jnp.float32),
            pltpu.VMEM((TPAD * C,), jnp.int32),
            pltpu.VMEM((TPAD,), jnp.float32),
            pltpu.VMEM((2 * N_EMBD * L,), jnp.float32),
            pltpu.VMEM((TPAD, DPAD), jnp.float32),
            pltpu.SemaphoreType.DMA,
            pltpu.SemaphoreType.DMA,
        ],
    )
    return f(x2, y2, tab_pad, wb_rep)


def kernel(x, y, table0, table1, table2, table3, table4, table5, table6,
           table7, table8, W, b):
    tab = jnp.concatenate(
        [table0, table1, table2, table3, table4, table5, table6, table7,
         table8], axis=0)
    tab_pad = jnp.pad(tab, ((0, 0), (0, TABW - N_EMBD))).reshape(-1)
    wb_rep = jnp.concatenate([
        jnp.repeat(W[0], L), jnp.repeat(b, L)])
    return _run(x.reshape(B, T * C), y[:, :, 0], tab_pad, wb_rep)


# speedup vs baseline: 9.4137x; 1.1708x over previous
import jax
import jax.numpy as jnp
from jax import lax
from jax.experimental import pallas as pl
from jax.experimental.pallas import tpu as pltpu
from jax.experimental.pallas import tpu_sc as plsc

B, T, C = 1024, 200, 7
N_EMBD = 32
LAG = 60
SIZES = (13, 32, 24, 7, 200, 2, 61, 2, 3)
OFFS = (0, 13, 45, 69, 76, 276, 278, 339, 341)
TOTAL_ROWS = 344
D = 10 * N_EMBD
TABW = N_EMBD + 1

NC, NS, L = 2, 16, 16
NW = NC * NS
ROWS_PER_W = B // NW
TPAD = 208
NG = TPAD // L
NG_LO = 7
NB = 25
NTC = 3
BAND_ROWS = NTC * 8
VROWS_LO = 14 * BAND_ROWS
VROWS_HI = 11 * BAND_ROWS


def _splat(v):
    return jnp.full((L,), v, jnp.int32)


def _iota():
    return lax.iota(jnp.int32, L)


def _sc_body(x_hbm, y_hbm, tab_hbm, wb_hbm, out_hbm,
             tab_v, x_v, y_v, wb_v, out_v, sem_lo, sem_hi):
    wid = lax.axis_index("s") * NC + lax.axis_index("c")
    base_row = wid * ROWS_PER_W

    pltpu.sync_copy(tab_hbm, tab_v)
    pltpu.sync_copy(wb_hbm, wb_v)

    def _store_row(g, tu, col0, vec):
        band_off = (tu // 8) * BAND_ROWS
        r_static = band_off + (col0 // 128) * 8 + (tu % 8)
        out_v[g * 2 * BAND_ROWS + r_static, pl.ds(col0 % 128, L)] = vec

    def _extract_i32(vec, tu):
        return jnp.sum(jnp.where(_iota() == tu, vec, 0))

    def _extract_f32(vec, tu):
        return jnp.sum(jnp.where(_iota() == tu, vec, jnp.float32(0)))

    def _fetch_row(r33):
        a0 = r33 + _iota()
        return (plsc.load_gather(tab_v, [a0]),
                plsc.load_gather(tab_v, [a0 + L]))

    @pl.loop(0, NG)
    def _const(g):
        tvec = g * L + _iota()
        r6 = (jnp.minimum(tvec, SIZES[6] - 1) + OFFS[6]) * TABW
        isfut = (tvec >= (T - LAG)).astype(jnp.int32)
        r7 = (isfut + OFFS[7]) * TABW
        r8 = (isfut + OFFS[8]) * TABW
        for tu in range(L):
            for p, rvec in ((6, r6), (7, r7), (8, r8)):
                v0, v1 = _fetch_row(_extract_i32(rvec, tu))
                _store_row(g, tu, p * N_EMBD, v0)
                _store_row(g, tu, p * N_EMBD + L, v1)

    def _compute_groups(g_lo, g_hi):
        @pl.loop(g_lo, g_hi)
        def _grp(g):
            tvec = g * L + _iota()
            xbase = tvec * C
            raws = [plsc.load_gather(x_v, [xbase + _splat(p + 1)])
                    for p in range(6)]
            rows = [(jnp.clip(raws[p], 0, SIZES[p] - 1) + OFFS[p]) * TABW
                    for p in range(6)]
            yvec = y_v[pl.ds(g * L, L)]
            wv0 = wb_v[pl.ds(0, L)]
            wv1 = wb_v[pl.ds(L, L)]
            bv0 = wb_v[pl.ds(2 * L, L)]
            bv1 = wb_v[pl.ds(3 * L, L)]
            for tu in range(L):
                r33 = [_extract_i32(rows[p], tu) for p in range(6)]
                for p in range(6):
                    v0, v1 = _fetch_row(r33[p])
                    _store_row(g, tu, p * N_EMBD, v0)
                    _store_row(g, tu, p * N_EMBD + L, v1)
                ysc = _extract_f32(yvec, tu)
                _store_row(g, tu, 9 * N_EMBD, ysc * wv0 + bv0)
                _store_row(g, tu, 9 * N_EMBD + L, ysc * wv1 + bv1)

    def _dma_lo(bi):
        return pltpu.make_async_copy(
            out_v.at[pl.ds(0, VROWS_LO)],
            out_hbm.at[bi, pl.ds(0, VROWS_LO)], sem_lo)

    def _dma_hi(bi):
        return pltpu.make_async_copy(
            out_v.at[pl.ds(VROWS_LO, VROWS_HI)],
            out_hbm.at[bi, pl.ds(VROWS_LO, VROWS_HI)], sem_hi)

    @pl.loop(0, ROWS_PER_W)
    def _row(j):
        bi = base_row + j
        pltpu.sync_copy(x_hbm.at[bi], x_v.at[pl.ds(0, T * C)])
        pltpu.sync_copy(y_hbm.at[bi], y_v.at[pl.ds(0, T)])

        @pl.when(j > 0)
        def _():
            _dma_lo(bi).wait()

        _compute_groups(0, NG_LO)
        _dma_lo(bi).start()

        @pl.when(j > 0)
        def _():
            _dma_hi(bi).wait()

        _compute_groups(NG_LO, NG)
        _dma_hi(bi).start()

    _dma_lo(base_row + ROWS_PER_W - 1).wait()
    _dma_hi(base_row + ROWS_PER_W - 1).wait()


@jax.jit
def _run(x2, y2, tab_pad, wb_rep):
    mesh = plsc.VectorSubcoreMesh(
        core_axis_name="c", subcore_axis_name="s",
        num_cores=NC, num_subcores=NS)
    f = pl.kernel(
        _sc_body,
        out_type=jax.ShapeDtypeStruct((B, NB * BAND_ROWS, 128),
                                      jnp.float32),
        mesh=mesh,
        compiler_params=pltpu.CompilerParams(
            needs_layout_passes=False, use_tc_tiling_on_sc=False),
        scratch_types=[
            pltpu.VMEM((TOTAL_ROWS * TABW,), jnp.float32),
            pltpu.VMEM((TPAD * C,), jnp.int32),
            pltpu.VMEM((TPAD,), jnp.float32),
            pltpu.VMEM((4 * L,), jnp.float32),
            pltpu.VMEM((2 * NG * BAND_ROWS, 128), jnp.float32),
            pltpu.SemaphoreType.DMA,
            pltpu.SemaphoreType.DMA,
        ],
    )
    return f(x2, y2, tab_pad, wb_rep)


def kernel(x, y, table0, table1, table2, table3, table4, table5, table6,
           table7, table8, W, b):
    tab = jnp.concatenate(
        [table0, table1, table2, table3, table4, table5, table6, table7,
         table8], axis=0)
    tab_pad = jnp.pad(tab, ((0, 0), (0, TABW - N_EMBD))).reshape(-1)
    wb_rep = jnp.concatenate([W[0], b])
    out = _run(x.reshape(B, T * C), y[:, :, 0], tab_pad, wb_rep)
    out = out.reshape(B, NB, NTC, 8, 128).transpose(0, 1, 3, 2, 4)
    return out.reshape(B, T, NTC * 128)[:, :, :D]
